# baseline (speedup 1.0000x reference)
import jax
import jax.numpy as jnp
from jax import lax
from jax.experimental import pallas as pl
from jax.experimental.pallas import tpu as pltpu
from jax.experimental.pallas import tpu_sc as plsc

_KVOL = 27
_NC, _NS = 2, 16
_NW = _NC * _NS
_CH = 720


def _mm_body(x_ref, w_ref, o_ref):
    o_ref[0] = jax.lax.dot_general(
        x_ref[...], w_ref[0],
        (((1,), (0,)), ((), ())),
        preferred_element_type=jnp.float32,
    )


def _cand_matmul(x, W):
    n, d = x.shape
    kvol, _, c_out = W.shape
    bn = 2000
    return pl.pallas_call(
        _mm_body,
        grid=(kvol, n // bn),
        in_specs=[
            pl.BlockSpec((bn, d), lambda k, i: (i, 0)),
            pl.BlockSpec((1, d, c_out), lambda k, i: (k, 0, 0)),
        ],
        out_specs=pl.BlockSpec((1, bn, c_out), lambda k, i: (k, i, 0)),
        out_shape=jax.ShapeDtypeStruct((kvol, n, c_out), jnp.float32),
    )(x, W)


def _lane_scalar(vec16, idx):
    lanes = lax.iota(jnp.int32, 16)
    return jnp.max(jnp.where(lanes == idx, vec16, jnp.iinfo(jnp.int32).min))


def _sc_body(cand_ref, sid_ref, segsc_ref, mask_ref, bias_ref, zsrc_ref,
             out_ref,
             zero_vm, mvm, sc16, sid24, idx16, rows_vm, acc_vm, bias_vm, sem):
    m_rows = out_ref.shape[0]
    nch = m_rows // _CH
    per = (nch + _NW - 1) // _NW
    wid = lax.axis_index("s") * _NC + lax.axis_index("c")

    pltpu.sync_copy(bias_ref, bias_vm)
    pltpu.sync_copy(zsrc_ref, zero_vm)

    def do_row(j):
        base = (j // 8) * 8
        pltpu.sync_copy(segsc_ref.at[pl.ds(2 * base, 16)], sc16)
        pair = sc16[...]
        d2 = 2 * (j - base)
        s = _lane_scalar(pair, d2)
        c = _lane_scalar(pair, d2 + 1)
        for oc in range(8):
            acc_vm[0, pl.ds(oc * 16, 16)] = bias_vm[0, pl.ds(oc * 16, 16)]

        def round_body(t, carry):
            p = s + t * 16
            ab = (p // 8) * 8
            d = p - ab
            pltpu.sync_copy(sid_ref.at[pl.ds(ab, 24)], sid24)
            lanes = lax.iota(jnp.int32, 16)
            ids = plsc.load_gather(sid24, [d + lanes])
            valid = (t * 16 + lanes) < c
            idx16[...] = jnp.where(valid, ids, 0)
            pltpu.async_copy(cand_ref.at[idx16], rows_vm, sem).wait()
            for r in range(16):
                @pl.when((t * 16 + r) < c)
                def _():
                    for oc in range(8):
                        acc_vm[0, pl.ds(oc * 16, 16)] = (
                            acc_vm[0, pl.ds(oc * 16, 16)]
                            + rows_vm[r, pl.ds(oc * 16, 16)])
            return carry

        lax.fori_loop(0, (c + 15) // 16, round_body, 0)
        pltpu.sync_copy(acc_vm, out_ref.at[pl.ds(j, 1)])

    def chunk_body(cid, carry):
        r0 = cid * _CH
        pltpu.sync_copy(mask_ref.at[pl.ds(r0, _CH)], mvm)
        pltpu.sync_copy(zero_vm, out_ref.at[pl.ds(r0, _CH)])

        def vec_body(v, carry2):
            m16 = mvm[pl.ds(v * 16, 16)]

            @pl.when(jnp.sum(m16) != 0)
            def _():
                def lane_body(lane, carry3):
                    @pl.when(_lane_scalar(m16, lane) > 0)
                    def _():
                        do_row(r0 + v * 16 + lane)
                    return carry3
                lax.fori_loop(0, 16, lane_body, 0)
            return carry2

        lax.fori_loop(0, _CH // 16, vec_body, 0)
        return carry

    c0 = wid * per
    lax.fori_loop(c0, jnp.minimum(nch, c0 + per), chunk_body, 0)


def _sc_output(cand2d, sid_pad, segsc, mask_i32, b):
    m_rows = cand2d.shape[0]
    c_out = cand2d.shape[1]
    return pl.kernel(
        _sc_body,
        out_type=jax.ShapeDtypeStruct((m_rows, c_out), jnp.float32),
        mesh=plsc.VectorSubcoreMesh(core_axis_name="c", subcore_axis_name="s"),
        scratch_types=[
            pltpu.VMEM((_CH, c_out), jnp.float32),
            pltpu.VMEM((_CH,), jnp.int32),
            pltpu.VMEM((16,), jnp.int32),
            pltpu.VMEM((24,), jnp.int32),
            pltpu.VMEM((16,), jnp.int32),
            pltpu.VMEM((16, c_out), jnp.float32),
            pltpu.VMEM((1, c_out), jnp.float32),
            pltpu.VMEM((1, c_out), jnp.float32),
            pltpu.SemaphoreType.DMA,
        ],
        compiler_params=pltpu.CompilerParams(needs_layout_passes=False),
    )(cand2d, sid_pad, segsc, mask_i32, b.reshape(1, c_out),
      jnp.zeros((_CH, c_out), jnp.float32))


def kernel(x, coords, ref_coords, W, b):
    n = x.shape[0]
    c_out = W.shape[2]
    M = n * _KVOL
    R = ref_coords.shape[0]
    T = M + R

    coords = coords.astype(jnp.int32)
    ref_coords = ref_coords.astype(jnp.int32)
    cmax = jnp.maximum(jnp.max(coords[:, 0]), 2 * jnp.max(coords[:, 1:4]) + 2)
    step = jnp.maximum(cmax, jnp.max(ref_coords)) + 1

    off = jnp.arange(_KVOL, dtype=jnp.int32)
    koff = (off % 3) + ((off // 3) % 3) * step + (off // 9) * step * step
    base = (2 * coords[:, 3] + 2 * coords[:, 2] * step
            + 2 * coords[:, 1] * step * step
            + coords[:, 0] * step * step * step)
    flat = (koff[:, None] + base[None, :]).reshape(-1)
    ref_flat = (ref_coords[:, 3] + ref_coords[:, 2] * step
                + ref_coords[:, 1] * step * step
                + ref_coords[:, 0] * step * step * step)

    keys = jnp.concatenate([flat, ref_flat])
    ids = jnp.arange(T, dtype=jnp.int32)
    sk, sid = lax.sort((keys, ids), num_keys=1)

    is_cand = sid < M
    is_start = jnp.concatenate(
        [jnp.ones((1,), jnp.bool_), sk[1:] != sk[:-1]])
    cand_new = is_start & is_cand
    seg_of = jnp.cumsum(cand_new.astype(jnp.int32)) - 1
    U = seg_of[-1] + 1
    run_id = jnp.cumsum(is_start.astype(jnp.int32)) - 1
    latest_cand_run = lax.cummax(jnp.where(cand_new, run_id, -1))
    ref_hit = (~is_cand) & (latest_cand_run == run_id)

    dump = M + (ids & 511)
    tgt = jnp.where(cand_new, seg_of, dump)
    uniq_raw = jnp.zeros(M + 512, jnp.int32).at[tgt].add(
        jnp.where(cand_new, sk, 0))
    st_raw = jnp.zeros(M + 512, jnp.int32).at[tgt].add(
        jnp.where(cand_new, ids, 0))
    cnt_raw = jnp.zeros(M + 512, jnp.int32).at[
        jnp.where(is_cand, seg_of, dump)].add(1)
    msk_raw = jnp.zeros(M + 512, jnp.int32).at[
        jnp.where(ref_hit, seg_of, dump)].add(1)

    fill = jnp.iinfo(jnp.int32).max
    uniq = jnp.where(jnp.arange(M) < U, uniq_raw[:M], fill)
    mask_i32 = msk_raw[:M]
    segsc = jnp.stack([st_raw[:M], cnt_raw[:M]], axis=1).reshape(-1)

    cand2d = _cand_matmul(x, W).reshape(M, c_out)
    sid_pad = jnp.concatenate([sid, jnp.zeros((32,), jnp.int32)])
    pruned = _sc_output(cand2d, sid_pad, segsc, mask_i32, b)
    return uniq, pruned, mask_i32 > 0


# speedup vs baseline: 1.0996x; 1.0996x over previous
import jax
import jax.numpy as jnp
from jax import lax
from jax.experimental import pallas as pl
from jax.experimental.pallas import tpu as pltpu
from jax.experimental.pallas import tpu_sc as plsc

_KVOL = 27
_NC, _NS = 2, 16
_NW = _NC * _NS
_CH = 720


def _mm_body(x_ref, w_ref, o_ref):
    o_ref[0] = jax.lax.dot_general(
        x_ref[...], w_ref[0],
        (((1,), (0,)), ((), ())),
        preferred_element_type=jnp.float32,
    )


def _cand_matmul(x, W):
    n, d = x.shape
    kvol, _, c_out = W.shape
    bn = 2000
    return pl.pallas_call(
        _mm_body,
        grid=(kvol, n // bn),
        in_specs=[
            pl.BlockSpec((bn, d), lambda k, i: (i, 0)),
            pl.BlockSpec((1, d, c_out), lambda k, i: (k, 0, 0)),
        ],
        out_specs=pl.BlockSpec((1, bn, c_out), lambda k, i: (k, i, 0)),
        out_shape=jax.ShapeDtypeStruct((kvol, n, c_out), jnp.float32),
    )(x, W)


def _lane_scalar(vec16, idx):
    lanes = lax.iota(jnp.int32, 16)
    return jnp.max(jnp.where(lanes == idx, vec16, jnp.iinfo(jnp.int32).min))


def _sc_body(cand_ref, sid_ref, segsc_ref, mask_ref, bias_ref, zsrc_ref,
             out_ref,
             zero_vm, mvm, sc16, sid24, idx16, rows_vm, acc_vm, bias_vm, sem):
    m_rows = out_ref.shape[0]
    nch = m_rows // _CH
    per = (nch + _NW - 1) // _NW
    wid = lax.axis_index("s") * _NC + lax.axis_index("c")

    pltpu.sync_copy(bias_ref, bias_vm)
    pltpu.sync_copy(zsrc_ref, zero_vm)

    def do_row(j):
        base = (j // 8) * 8
        pltpu.sync_copy(segsc_ref.at[pl.ds(2 * base, 16)], sc16)
        pair = sc16[...]
        d2 = 2 * (j - base)
        s = _lane_scalar(pair, d2)
        c = _lane_scalar(pair, d2 + 1)
        for oc in range(8):
            acc_vm[0, pl.ds(oc * 16, 16)] = bias_vm[0, pl.ds(oc * 16, 16)]

        def round_body(t, carry):
            p = s + t * 16
            ab = (p // 8) * 8
            d = p - ab
            pltpu.sync_copy(sid_ref.at[pl.ds(ab, 24)], sid24)
            lanes = lax.iota(jnp.int32, 16)
            ids = plsc.load_gather(sid24, [d + lanes])
            valid = (t * 16 + lanes) < c
            idx16[...] = jnp.where(valid, ids, 0)
            pltpu.async_copy(cand_ref.at[idx16], rows_vm, sem).wait()
            for r in range(16):
                @pl.when((t * 16 + r) < c)
                def _():
                    for oc in range(8):
                        acc_vm[0, pl.ds(oc * 16, 16)] = (
                            acc_vm[0, pl.ds(oc * 16, 16)]
                            + rows_vm[r, pl.ds(oc * 16, 16)])
            return carry

        lax.fori_loop(0, (c + 15) // 16, round_body, 0)
        pltpu.sync_copy(acc_vm, out_ref.at[pl.ds(j, 1)])

    def chunk_body(cid, carry):
        r0 = cid * _CH
        pltpu.sync_copy(mask_ref.at[pl.ds(r0, _CH)], mvm)
        pltpu.sync_copy(zero_vm, out_ref.at[pl.ds(r0, _CH)])

        def vec_body(v, carry2):
            m16 = mvm[pl.ds(v * 16, 16)]

            @pl.when(jnp.sum(m16) != 0)
            def _():
                def lane_body(lane, carry3):
                    @pl.when(_lane_scalar(m16, lane) > 0)
                    def _():
                        do_row(r0 + v * 16 + lane)
                    return carry3
                lax.fori_loop(0, 16, lane_body, 0)
            return carry2

        lax.fori_loop(0, _CH // 16, vec_body, 0)
        return carry

    c0 = wid * per
    lax.fori_loop(c0, jnp.minimum(nch, c0 + per), chunk_body, 0)


def _sc_output(cand2d, sid_pad, segsc, mask_i32, b):
    m_rows = cand2d.shape[0]
    c_out = cand2d.shape[1]
    return pl.kernel(
        _sc_body,
        out_type=jax.ShapeDtypeStruct((m_rows, c_out), jnp.float32),
        mesh=plsc.VectorSubcoreMesh(core_axis_name="c", subcore_axis_name="s"),
        scratch_types=[
            pltpu.VMEM((_CH, c_out), jnp.float32),
            pltpu.VMEM((_CH,), jnp.int32),
            pltpu.VMEM((16,), jnp.int32),
            pltpu.VMEM((24,), jnp.int32),
            pltpu.VMEM((16,), jnp.int32),
            pltpu.VMEM((16, c_out), jnp.float32),
            pltpu.VMEM((1, c_out), jnp.float32),
            pltpu.VMEM((1, c_out), jnp.float32),
            pltpu.SemaphoreType.DMA,
        ],
        compiler_params=pltpu.CompilerParams(needs_layout_passes=False),
    )(cand2d, sid_pad, segsc, mask_i32, b.reshape(1, c_out),
      jnp.zeros((_CH, c_out), jnp.float32))


def kernel(x, coords, ref_coords, W, b):
    n = x.shape[0]
    c_out = W.shape[2]
    M = n * _KVOL
    R = ref_coords.shape[0]
    T = M + R

    coords = coords.astype(jnp.int32)
    ref_coords = ref_coords.astype(jnp.int32)
    cmax = jnp.maximum(jnp.max(coords[:, 0]), 2 * jnp.max(coords[:, 1:4]) + 2)
    step = jnp.maximum(cmax, jnp.max(ref_coords)) + 1

    off = jnp.arange(_KVOL, dtype=jnp.int32)
    koff = (off % 3) + ((off // 3) % 3) * step + (off // 9) * step * step
    base = (2 * coords[:, 3] + 2 * coords[:, 2] * step
            + 2 * coords[:, 1] * step * step
            + coords[:, 0] * step * step * step)
    flat = (koff[:, None] + base[None, :]).reshape(-1)
    ref_flat = (ref_coords[:, 3] + ref_coords[:, 2] * step
                + ref_coords[:, 1] * step * step
                + ref_coords[:, 0] * step * step * step)

    keys = jnp.concatenate([flat, ref_flat])
    ids = jnp.arange(T, dtype=jnp.int32)
    sk, sid = lax.sort((keys, ids), num_keys=1)

    is_cand = sid < M
    is_start = jnp.concatenate(
        [jnp.ones((1,), jnp.bool_), sk[1:] != sk[:-1]])
    cand_new = is_start & is_cand
    seg_of = jnp.cumsum(cand_new.astype(jnp.int32)) - 1
    U = seg_of[-1] + 1
    run_id = jnp.cumsum(is_start.astype(jnp.int32)) - 1
    latest_cand_run = lax.cummax(jnp.where(cand_new, run_id, -1))
    ref_hit = (~is_cand) & (latest_cand_run == run_id)

    dump = M + (ids & 511)
    tgt = jnp.where(cand_new, seg_of, dump)
    uniq_raw = jnp.zeros(M + 512, jnp.int32).at[tgt].add(
        jnp.where(cand_new, sk, 0))
    st_raw = jnp.zeros(M + 512, jnp.int32).at[tgt].add(
        jnp.where(cand_new, ids, 0))
    cnt_raw = jnp.zeros(M + 512, jnp.int32).at[
        jnp.where(is_cand, seg_of, dump)].add(1)
    msk_raw = jnp.zeros(M + 512, jnp.int32).at[
        jnp.where(ref_hit, seg_of, dump)].add(1)

    fill = jnp.iinfo(jnp.int32).max
    uniq = jnp.where(jnp.arange(M) < U, uniq_raw[:M], fill)
    mask_i32 = msk_raw[:M]
    segsc = jnp.stack([st_raw[:M], cnt_raw[:M]], axis=1).reshape(-1)

    cand2d = _cand_matmul(x, W).reshape(M, c_out)
    sid_pad = jnp.concatenate([sid, jnp.zeros((32,), jnp.int32)])
    pruned = _sc_output(cand2d, sid_pad, segsc, mask_i32, b)
    import os as _os
    stage = _os.environ.get("STAGE", "")
    if stage == "sort":
        return sk[:M], jnp.zeros((M, c_out), jnp.float32), jnp.zeros((M,), jnp.bool_)
    if stage == "scan":
        return uniq, jnp.zeros((M, c_out), jnp.float32), mask_i32 > 0
    if stage == "noeinsum":
        pruned2 = _sc_output(jnp.zeros((M, c_out), jnp.float32), sid_pad, segsc, mask_i32, b)
        return uniq, pruned2, mask_i32 > 0
    return uniq, pruned, mask_i32 > 0


# speedup vs baseline: 1.1140x; 1.0131x over previous
import jax
import jax.numpy as jnp
from jax import lax
from jax.experimental import pallas as pl
from jax.experimental.pallas import tpu as pltpu
from jax.experimental.pallas import tpu_sc as plsc

_KVOL = 27
_NC, _NS = 2, 16
_NW = _NC * _NS
_CH = 720


def _mm_body(x_ref, w_ref, o_ref):
    o_ref[0] = jax.lax.dot_general(
        x_ref[...], w_ref[0],
        (((1,), (0,)), ((), ())),
        preferred_element_type=jnp.float32,
    )


def _cand_matmul(x, W):
    n, d = x.shape
    kvol, _, c_out = W.shape
    bn = 2000
    return pl.pallas_call(
        _mm_body,
        grid=(kvol, n // bn),
        in_specs=[
            pl.BlockSpec((bn, d), lambda k, i: (i, 0)),
            pl.BlockSpec((1, d, c_out), lambda k, i: (k, 0, 0)),
        ],
        out_specs=pl.BlockSpec((1, bn, c_out), lambda k, i: (k, i, 0)),
        out_shape=jax.ShapeDtypeStruct((kvol, n, c_out), jnp.float32),
    )(x, W)


def _lane_scalar(vec16, idx):
    lanes = lax.iota(jnp.int32, 16)
    return jnp.max(jnp.where(lanes == idx, vec16, jnp.iinfo(jnp.int32).min))


def _sc_body(cand_ref, sid_ref, segsc_ref, mask_ref, bias_ref, zsrc_ref,
             out_ref,
             zero_vm, mvm, sc16, sid24, idx16, rows_vm, acc_vm, bias_vm, sem):
    m_rows = out_ref.shape[0]
    nch = m_rows // _CH
    per = (nch + _NW - 1) // _NW
    wid = lax.axis_index("s") * _NC + lax.axis_index("c")

    pltpu.sync_copy(bias_ref, bias_vm)
    pltpu.sync_copy(zsrc_ref, zero_vm)

    def do_row(j):
        base = (j // 8) * 8
        pltpu.sync_copy(segsc_ref.at[pl.ds(2 * base, 16)], sc16)
        pair = sc16[...]
        d2 = 2 * (j - base)
        s = _lane_scalar(pair, d2)
        c = _lane_scalar(pair, d2 + 1)
        for oc in range(8):
            acc_vm[0, pl.ds(oc * 16, 16)] = bias_vm[0, pl.ds(oc * 16, 16)]

        def round_body(t, carry):
            p = s + t * 16
            ab = (p // 8) * 8
            d = p - ab
            pltpu.sync_copy(sid_ref.at[pl.ds(ab, 24)], sid24)
            lanes = lax.iota(jnp.int32, 16)
            ids = plsc.load_gather(sid24, [d + lanes])
            valid = (t * 16 + lanes) < c
            idx16[...] = jnp.where(valid, ids, 0)
            pltpu.async_copy(cand_ref.at[idx16], rows_vm, sem).wait()
            for r in range(16):
                @pl.when((t * 16 + r) < c)
                def _():
                    for oc in range(8):
                        acc_vm[0, pl.ds(oc * 16, 16)] = (
                            acc_vm[0, pl.ds(oc * 16, 16)]
                            + rows_vm[r, pl.ds(oc * 16, 16)])
            return carry

        lax.fori_loop(0, (c + 15) // 16, round_body, 0)
        pltpu.sync_copy(acc_vm, out_ref.at[pl.ds(j, 1)])

    def chunk_body(cid, carry):
        r0 = cid * _CH
        pltpu.sync_copy(mask_ref.at[pl.ds(r0, _CH)], mvm)
        pltpu.sync_copy(zero_vm, out_ref.at[pl.ds(r0, _CH)])

        def vec_body(v, carry2):
            m16 = mvm[pl.ds(v * 16, 16)]

            @pl.when(jnp.sum(m16) != 0)
            def _():
                def lane_body(lane, carry3):
                    @pl.when(_lane_scalar(m16, lane) > 0)
                    def _():
                        do_row(r0 + v * 16 + lane)
                    return carry3
                lax.fori_loop(0, 16, lane_body, 0)
            return carry2

        lax.fori_loop(0, _CH // 16, vec_body, 0)
        return carry

    c0 = wid * per
    lax.fori_loop(c0, jnp.minimum(nch, c0 + per), chunk_body, 0)


def _sc_output(cand2d, sid_pad, segsc, mask_i32, b):
    m_rows = cand2d.shape[0]
    c_out = cand2d.shape[1]
    return pl.kernel(
        _sc_body,
        out_type=jax.ShapeDtypeStruct((m_rows, c_out), jnp.float32),
        mesh=plsc.VectorSubcoreMesh(core_axis_name="c", subcore_axis_name="s"),
        scratch_types=[
            pltpu.VMEM((_CH, c_out), jnp.float32),
            pltpu.VMEM((_CH,), jnp.int32),
            pltpu.VMEM((16,), jnp.int32),
            pltpu.VMEM((24,), jnp.int32),
            pltpu.VMEM((16,), jnp.int32),
            pltpu.VMEM((16, c_out), jnp.float32),
            pltpu.VMEM((1, c_out), jnp.float32),
            pltpu.VMEM((1, c_out), jnp.float32),
            pltpu.SemaphoreType.DMA,
        ],
        compiler_params=pltpu.CompilerParams(needs_layout_passes=False),
    )(cand2d, sid_pad, segsc, mask_i32, b.reshape(1, c_out),
      jnp.zeros((_CH, c_out), jnp.float32))


def kernel(x, coords, ref_coords, W, b):
    n = x.shape[0]
    c_out = W.shape[2]
    M = n * _KVOL
    R = ref_coords.shape[0]
    T = M + R

    coords = coords.astype(jnp.int32)
    ref_coords = ref_coords.astype(jnp.int32)
    cmax = jnp.maximum(jnp.max(coords[:, 0]), 2 * jnp.max(coords[:, 1:4]) + 2)
    step = jnp.maximum(cmax, jnp.max(ref_coords)) + 1

    off = jnp.arange(_KVOL, dtype=jnp.int32)
    koff = (off % 3) + ((off // 3) % 3) * step + (off // 9) * step * step
    base = (2 * coords[:, 3] + 2 * coords[:, 2] * step
            + 2 * coords[:, 1] * step * step
            + coords[:, 0] * step * step * step)
    flat = (koff[:, None] + base[None, :]).reshape(-1)
    ref_flat = (ref_coords[:, 3] + ref_coords[:, 2] * step
                + ref_coords[:, 1] * step * step
                + ref_coords[:, 0] * step * step * step)

    keys = jnp.concatenate([flat * 2, ref_flat * 2 + 1])
    ids = jnp.arange(T, dtype=jnp.int32)
    sk, sid = lax.sort((keys, ids), num_keys=1, is_stable=False)
    sk = sk >> 1

    is_cand = sid < M
    is_start = jnp.concatenate(
        [jnp.ones((1,), jnp.bool_), sk[1:] != sk[:-1]])
    cand_new = is_start & is_cand
    seg_of = jnp.cumsum(cand_new.astype(jnp.int32)) - 1
    U = seg_of[-1] + 1
    run_id = jnp.cumsum(is_start.astype(jnp.int32)) - 1
    latest_cand_run = lax.cummax(jnp.where(cand_new, run_id, -1))
    ref_hit = (~is_cand) & (latest_cand_run == run_id)

    dump = M + (ids & 511)
    tgt = jnp.where(cand_new, seg_of, dump)
    uniq_raw = jnp.zeros(M + 512, jnp.int32).at[tgt].add(
        jnp.where(cand_new, sk, 0))
    st_raw = jnp.zeros(M + 512, jnp.int32).at[tgt].add(
        jnp.where(cand_new, ids, 0))
    cnt_raw = jnp.zeros(M + 512, jnp.int32).at[
        jnp.where(is_cand, seg_of, dump)].add(1)
    msk_raw = jnp.zeros(M + 512, jnp.int32).at[
        jnp.where(ref_hit, seg_of, dump)].add(1)

    fill = jnp.iinfo(jnp.int32).max
    uniq = jnp.where(jnp.arange(M) < U, uniq_raw[:M], fill)
    mask_i32 = msk_raw[:M]
    segsc = jnp.stack([st_raw[:M], cnt_raw[:M]], axis=1).reshape(-1)

    cand2d = _cand_matmul(x, W).reshape(M, c_out)
    sid_pad = jnp.concatenate([sid, jnp.zeros((32,), jnp.int32)])
    pruned = _sc_output(cand2d, sid_pad, segsc, mask_i32, b)
    return uniq, pruned, mask_i32 > 0


# speedup vs baseline: 1.1150x; 1.0009x over previous
import jax
import jax.numpy as jnp
from jax import lax
from jax.experimental import pallas as pl
from jax.experimental.pallas import tpu as pltpu
from jax.experimental.pallas import tpu_sc as plsc

_KVOL = 27
_NC, _NS = 2, 16
_NW = _NC * _NS
_CH = 720


def _mm_body(x_ref, w_ref, o_ref):
    o_ref[0] = jax.lax.dot_general(
        x_ref[...], w_ref[0],
        (((1,), (0,)), ((), ())),
        preferred_element_type=jnp.float32,
    )


def _cand_matmul(x, W):
    n, d = x.shape
    kvol, _, c_out = W.shape
    bn = 2000
    return pl.pallas_call(
        _mm_body,
        grid=(kvol, n // bn),
        in_specs=[
            pl.BlockSpec((bn, d), lambda k, i: (i, 0)),
            pl.BlockSpec((1, d, c_out), lambda k, i: (k, 0, 0)),
        ],
        out_specs=pl.BlockSpec((1, bn, c_out), lambda k, i: (k, i, 0)),
        out_shape=jax.ShapeDtypeStruct((kvol, n, c_out), jnp.float32),
    )(x, W)


def _lane_scalar(vec16, idx):
    lanes = lax.iota(jnp.int32, 16)
    return jnp.max(jnp.where(lanes == idx, vec16, jnp.iinfo(jnp.int32).min))


def _sc_body(cand_ref, sid_ref, segsc_ref, mask_ref, bias_ref, zsrc_ref,
             out_ref,
             zero_vm, mvm, sc16, sid24, idx16, rows_vm, acc_vm, bias_vm, sem):
    m_rows = out_ref.shape[0]
    nch = m_rows // _CH
    per = (nch + _NW - 1) // _NW
    wid = lax.axis_index("s") * _NC + lax.axis_index("c")

    pltpu.sync_copy(bias_ref, bias_vm)
    pltpu.sync_copy(zsrc_ref, zero_vm)

    def do_row(j):
        base = (j // 8) * 8
        pltpu.sync_copy(segsc_ref.at[pl.ds(2 * base, 16)], sc16)
        pair = sc16[...]
        d2 = 2 * (j - base)
        s = _lane_scalar(pair, d2)
        c = _lane_scalar(pair, d2 + 1)
        for oc in range(8):
            acc_vm[0, pl.ds(oc * 16, 16)] = bias_vm[0, pl.ds(oc * 16, 16)]

        def round_body(t, carry):
            p = s + t * 16
            ab = (p // 8) * 8
            d = p - ab
            pltpu.sync_copy(sid_ref.at[pl.ds(ab, 24)], sid24)
            lanes = lax.iota(jnp.int32, 16)
            ids = plsc.load_gather(sid24, [d + lanes])
            valid = (t * 16 + lanes) < c
            idx16[...] = jnp.where(valid, ids, 0)
            pltpu.async_copy(cand_ref.at[idx16], rows_vm, sem).wait()
            for r in range(16):
                @pl.when((t * 16 + r) < c)
                def _():
                    for oc in range(8):
                        acc_vm[0, pl.ds(oc * 16, 16)] = (
                            acc_vm[0, pl.ds(oc * 16, 16)]
                            + rows_vm[r, pl.ds(oc * 16, 16)])
            return carry

        lax.fori_loop(0, (c + 15) // 16, round_body, 0)
        pltpu.sync_copy(acc_vm, out_ref.at[pl.ds(j, 1)])

    def chunk_body(cid, carry):
        r0 = cid * _CH
        pltpu.sync_copy(mask_ref.at[pl.ds(r0, _CH)], mvm)
        pltpu.sync_copy(zero_vm, out_ref.at[pl.ds(r0, _CH)])

        def vec_body(v, carry2):
            m16 = mvm[pl.ds(v * 16, 16)]

            @pl.when(jnp.sum(m16) != 0)
            def _():
                def lane_body(lane, carry3):
                    @pl.when(_lane_scalar(m16, lane) > 0)
                    def _():
                        do_row(r0 + v * 16 + lane)
                    return carry3
                lax.fori_loop(0, 16, lane_body, 0)
            return carry2

        lax.fori_loop(0, _CH // 16, vec_body, 0)
        return carry

    c0 = wid * per
    lax.fori_loop(c0, jnp.minimum(nch, c0 + per), chunk_body, 0)


def _sc_output(cand2d, sid_pad, segsc, mask_i32, b):
    m_rows = cand2d.shape[0]
    c_out = cand2d.shape[1]
    return pl.kernel(
        _sc_body,
        out_type=jax.ShapeDtypeStruct((m_rows, c_out), jnp.float32),
        mesh=plsc.VectorSubcoreMesh(core_axis_name="c", subcore_axis_name="s"),
        scratch_types=[
            pltpu.VMEM((_CH, c_out), jnp.float32),
            pltpu.VMEM((_CH,), jnp.int32),
            pltpu.VMEM((16,), jnp.int32),
            pltpu.VMEM((24,), jnp.int32),
            pltpu.VMEM((16,), jnp.int32),
            pltpu.VMEM((16, c_out), jnp.float32),
            pltpu.VMEM((1, c_out), jnp.float32),
            pltpu.VMEM((1, c_out), jnp.float32),
            pltpu.SemaphoreType.DMA,
        ],
        compiler_params=pltpu.CompilerParams(needs_layout_passes=False),
    )(cand2d, sid_pad, segsc, mask_i32, b.reshape(1, c_out),
      jnp.zeros((_CH, c_out), jnp.float32))


def kernel(x, coords, ref_coords, W, b):
    n = x.shape[0]
    c_out = W.shape[2]
    M = n * _KVOL
    R = ref_coords.shape[0]
    T = M + R

    coords = coords.astype(jnp.int32)
    ref_coords = ref_coords.astype(jnp.int32)
    cmax = jnp.maximum(jnp.max(coords[:, 0]), 2 * jnp.max(coords[:, 1:4]) + 2)
    step = jnp.maximum(cmax, jnp.max(ref_coords)) + 1

    off = jnp.arange(_KVOL, dtype=jnp.int32)
    koff = (off % 3) + ((off // 3) % 3) * step + (off // 9) * step * step
    base = (2 * coords[:, 3] + 2 * coords[:, 2] * step
            + 2 * coords[:, 1] * step * step
            + coords[:, 0] * step * step * step)
    flat = (koff[:, None] + base[None, :]).reshape(-1)
    ref_flat = (ref_coords[:, 3] + ref_coords[:, 2] * step
                + ref_coords[:, 1] * step * step
                + ref_coords[:, 0] * step * step * step)

    keys = jnp.concatenate([flat * 2, ref_flat * 2 + 1])
    ids = jnp.arange(T, dtype=jnp.int32)
    sk2, sid = lax.sort((keys, ids), num_keys=1, is_stable=False)
    sk = sk2 >> 1

    is_cand = (sk2 & 1) == 0
    new2 = jnp.concatenate([jnp.ones((1,), jnp.bool_), sk2[1:] != sk2[:-1]])
    samek = jnp.concatenate([jnp.zeros((1,), jnp.bool_), sk[1:] == sk[:-1]])
    cand_new = is_cand & ~samek
    seg_of = jnp.cumsum(cand_new.astype(jnp.int32)) - 1
    U = seg_of[-1] + 1
    ref_hit = (~is_cand) & new2 & samek

    dump = M + (ids & 511)
    tgt = jnp.where(cand_new, seg_of, dump)
    uniq_raw = jnp.zeros(M + 512, jnp.int32).at[tgt].add(
        jnp.where(cand_new, sk, 0))
    st_raw = jnp.zeros(M + 512, jnp.int32).at[tgt].add(
        jnp.where(cand_new, ids, 0))
    cnt_raw = jnp.zeros(M + 512, jnp.int32).at[
        jnp.where(is_cand, seg_of, dump)].add(1)
    msk_raw = jnp.zeros(M + 512, jnp.int32).at[
        jnp.where(ref_hit, seg_of, dump)].add(1)

    fill = jnp.iinfo(jnp.int32).max
    uniq = jnp.where(jnp.arange(M) < U, uniq_raw[:M], fill)
    mask_i32 = msk_raw[:M]
    segsc = jnp.stack([st_raw[:M], cnt_raw[:M]], axis=1).reshape(-1)

    cand2d = _cand_matmul(x, W).reshape(M, c_out)
    sid_pad = jnp.concatenate([sid, jnp.zeros((32,), jnp.int32)])
    pruned = _sc_output(cand2d, sid_pad, segsc, mask_i32, b)
    return uniq, pruned, mask_i32 > 0
